# baseline (speedup 1.0000x reference)
import functools

import jax
import jax.numpy as jnp
from jax import lax
from jax.experimental import pallas as pl
from jax.experimental.pallas import tpu as pltpu
from jax.experimental.pallas import tpu_sc as plsc

B = 16384
L = 20
LH = L // 2
EMB = 32
VOCAB1 = 100001
VPAD = 100008
NAMEF = EMB * VOCAB1
TVOCAB = 10000
NC = 2
NS = 16
NW = NC * NS

P1_CH = 2048
N1CH = B // P1_CH
CG = 4
CPG = EMB // CG
RG = NW // CG
RPG = B // RG
P2_CH = 256
N2CH = RPG // P2_CH


def _tree_sum(vs):
    while len(vs) > 1:
        nxt = [vs[i] + vs[i + 1] for i in range(0, len(vs) - 1, 2)]
        if len(vs) % 2:
            nxt.append(vs[-1])
        vs = nxt
    return vs[0]


def _body(aidx_hbm, idsT_hbm, nameF_hbm, textT_hbm, outT_hbm,
          big_v, ids_v, aidx_v, oute1_v, acc_v,
          isem0, isem1, osem0, osem1, bsem):
    wid = lax.axis_index("s") * NC + lax.axis_index("c")
    isems = [isem0, isem1]
    osems = [osem0, osem1]

    g = wid % CG
    h = wid // CG
    rbase = h * RPG
    cbase = g * CPG

    tds = [pltpu.async_copy(
        textT_hbm.at[cbase + j],
        big_v.at[pl.ds(j * TVOCAB, TVOCAB)], bsem)
        for j in range(CPG)]
    for d in tds:
        d.wait()

    lf = jnp.full((16,), float(L), dtype=jnp.float32)
    onef = jnp.full((16,), 1.0, dtype=jnp.float32)
    cvecs = [jnp.full((16,), j * TVOCAB, dtype=jnp.int32) for j in range(CPG)]
    t0s = [big_v[pl.ds(j * TVOCAB, 16)][0] for j in range(CPG)]

    off = wid * VOCAB1
    sh = lax.rem(off, 8)
    astart = pl.multiple_of(off - sh, 8)
    shv = jnp.full((16,), 0, jnp.int32) + sh
    big_dma = None

    idmas = [pltpu.async_copy(
        idsT_hbm.at[:, pl.ds(rbase + p * P2_CH, P2_CH)], ids_v.at[p],
        isems[p]) for p in range(2)]
    odmas = [None, None]
    for ch in range(N2CH):
        p = ch % 2
        r0 = rbase + ch * P2_CH
        idmas[p].wait()
        if odmas[p] is not None:
            for d in odmas[p]:
                d.wait()

        @plsc.parallel_loop(0, P2_CH // 16, unroll=1)
        def e2grp(gg):
            rr = gg * 16
            iv0 = [ids_v[p, t, pl.ds(rr, 16)] for t in range(LH)]
            iv1 = [ids_v[p, t, pl.ds(rr, 16)] for t in range(LH, L)]
            cnt = (_tree_sum([jnp.where(v != 0, 1.0, 0.0) for v in iv0])
                   + _tree_sum([jnp.where(v != 0, 1.0, 0.0) for v in iv1]))
            n0f = lf - cnt
            recip = 1.0 / jnp.maximum(cnt, onef)
            for j in range(CPG):
                acc = (_tree_sum([plsc.load_gather(big_v, [v + cvecs[j]])
                                  for v in iv0])
                       + _tree_sum([plsc.load_gather(big_v, [v + cvecs[j]])
                                    for v in iv1]))
                acc_v[p, j, pl.ds(rr, 16)] = (acc - n0f * t0s[j]) * recip

        if ch == N2CH - 1:
            big_dma = pltpu.async_copy(
                nameF_hbm.at[pl.ds(astart, VPAD)], big_v, bsem)
        odmas[p] = [pltpu.async_copy(
            acc_v.at[p, j],
            outT_hbm.at[EMB + cbase + j, pl.ds(r0, P2_CH)], osems[p])
            for j in range(CPG)]
        if ch + 2 < N2CH:
            idmas[p] = pltpu.async_copy(
                idsT_hbm.at[:, pl.ds(rbase + (ch + 2) * P2_CH, P2_CH)],
                ids_v.at[p], isems[p])
    for ds in odmas:
        for d in ds:
            d.wait()

    idmas = [pltpu.async_copy(
        aidx_hbm.at[pl.ds(p * P1_CH, P1_CH)], aidx_v.at[p], isems[p])
        for p in range(2)]
    big_dma.wait()
    odmas = [None, None]
    for ch in range(N1CH):
        p = ch % 2
        idmas[p].wait()
        if odmas[p] is not None:
            odmas[p].wait()

        @plsc.parallel_loop(0, P1_CH // 16, unroll=8)
        def e1grp(g):
            rr = g * 16
            idx = aidx_v[p, pl.ds(rr, 16)] + shv
            oute1_v[p, pl.ds(rr, 16)] = plsc.load_gather(big_v, [idx])

        odmas[p] = pltpu.async_copy(
            oute1_v.at[p], outT_hbm.at[wid, pl.ds(ch * P1_CH, P1_CH)],
            osems[p])
        if ch + 2 < N1CH:
            idmas[p] = pltpu.async_copy(
                aidx_hbm.at[pl.ds((ch + 2) * P1_CH, P1_CH)], aidx_v.at[p],
                isems[p])
    for d in odmas:
        d.wait()


def kernel(anime_ids, token_ids, name_table, text_table):
    aidx = anime_ids.astype(jnp.int32)
    idsT = token_ids.astype(jnp.int32).T
    nameF = name_table.T.reshape(NAMEF)
    textT = text_table.T

    mesh = plsc.VectorSubcoreMesh(core_axis_name="c", subcore_axis_name="s")
    run = functools.partial(
        pl.kernel, mesh=mesh,
        out_type=jax.ShapeDtypeStruct((2 * EMB, B), jnp.float32),
        compiler_params=pltpu.CompilerParams(
            needs_layout_passes=False, use_tc_tiling_on_sc=False),
        scratch_types=[
            pltpu.VMEM((VPAD,), jnp.float32),
            pltpu.VMEM((2, L, P2_CH), jnp.int32),
            pltpu.VMEM((2, P1_CH), jnp.int32),
            pltpu.VMEM((2, P1_CH), jnp.float32),
            pltpu.VMEM((2, CPG, P2_CH), jnp.float32),
            pltpu.SemaphoreType.DMA,
            pltpu.SemaphoreType.DMA,
            pltpu.SemaphoreType.DMA,
            pltpu.SemaphoreType.DMA,
            pltpu.SemaphoreType.DMA,
        ],
    )(_body)
    outT = run(aidx, idsT, nameF, textT)
    return outT.T


# speedup vs baseline: 1.0137x; 1.0137x over previous
import functools

import jax
import jax.numpy as jnp
from jax import lax
from jax.experimental import pallas as pl
from jax.experimental.pallas import tpu as pltpu
from jax.experimental.pallas import tpu_sc as plsc

B = 16384
L = 20
LH = L // 2
EMB = 32
VOCAB1 = 100001
VPAD = 100008
NAMEF = EMB * VOCAB1
TVOCAB = 10000
NC = 2
NS = 16
NW = NC * NS

P1_CH = 2048
N1CH = B // P1_CH
CG = 4
CPG = EMB // CG
RG = NW // CG
RPG = B // RG
P2_CH = 256
N2CH = RPG // P2_CH


def _tree_sum(vs):
    while len(vs) > 1:
        nxt = [vs[i] + vs[i + 1] for i in range(0, len(vs) - 1, 2)]
        if len(vs) % 2:
            nxt.append(vs[-1])
        vs = nxt
    return vs[0]


def _body(aidx_hbm, idsT_hbm, nameF_hbm, textT_hbm, outT_hbm,
          big_v, ids_v, aidx_v, oute1_v, acc_v,
          isem0, isem1, osem0, osem1, bsem):
    wid = lax.axis_index("s") * NC + lax.axis_index("c")

    off = wid * VOCAB1
    sh = lax.rem(off, 8)
    astart = pl.multiple_of(off - sh, 8)
    big_dma = pltpu.async_copy(nameF_hbm.at[pl.ds(astart, VPAD)], big_v, bsem)
    shv = jnp.full((16,), 0, jnp.int32) + sh

    isems = [isem0, isem1]
    osems = [osem0, osem1]
    idmas = [pltpu.async_copy(
        aidx_hbm.at[pl.ds(p * P1_CH, P1_CH)], aidx_v.at[p], isems[p])
        for p in range(2)]
    big_dma.wait()
    odmas = [None, None]
    for ch in range(N1CH):
        p = ch % 2
        idmas[p].wait()
        if odmas[p] is not None:
            odmas[p].wait()

        @plsc.parallel_loop(0, P1_CH // 16, unroll=8)
        def e1grp(g):
            rr = g * 16
            idx = aidx_v[p, pl.ds(rr, 16)] + shv
            oute1_v[p, pl.ds(rr, 16)] = plsc.load_gather(big_v, [idx])

        odmas[p] = pltpu.async_copy(
            oute1_v.at[p], outT_hbm.at[wid, pl.ds(ch * P1_CH, P1_CH)],
            osems[p])
        if ch + 2 < N1CH:
            idmas[p] = pltpu.async_copy(
                aidx_hbm.at[pl.ds((ch + 2) * P1_CH, P1_CH)], aidx_v.at[p],
                isems[p])
    for d in odmas:
        d.wait()

    g = wid % CG
    h = wid // CG
    rbase = h * RPG
    cbase = g * CPG

    tds = [pltpu.async_copy(
        textT_hbm.at[cbase + j],
        big_v.at[pl.ds(j * TVOCAB, TVOCAB)], bsem)
        for j in range(CPG)]
    for d in tds:
        d.wait()

    lf = jnp.full((16,), float(L), dtype=jnp.float32)
    onef = jnp.full((16,), 1.0, dtype=jnp.float32)
    cvecs = [jnp.full((16,), j * TVOCAB, dtype=jnp.int32) for j in range(CPG)]
    t0s = [big_v[pl.ds(j * TVOCAB, 16)][0] for j in range(CPG)]

    idmas = [pltpu.async_copy(
        idsT_hbm.at[:, pl.ds(rbase + p * P2_CH, P2_CH)], ids_v.at[p],
        isems[p]) for p in range(2)]
    odmas = [None, None]
    for ch in range(N2CH):
        p = ch % 2
        r0 = rbase + ch * P2_CH
        idmas[p].wait()
        if odmas[p] is not None:
            for d in odmas[p]:
                d.wait()

        @plsc.parallel_loop(0, P2_CH // 16, unroll=1)
        def e2grp(gg):
            rr = gg * 16
            iv0 = [ids_v[p, t, pl.ds(rr, 16)] for t in range(LH)]
            iv1 = [ids_v[p, t, pl.ds(rr, 16)] for t in range(LH, L)]
            cnt = (_tree_sum([jnp.where(v != 0, 1.0, 0.0) for v in iv0])
                   + _tree_sum([jnp.where(v != 0, 1.0, 0.0) for v in iv1]))
            n0f = lf - cnt
            recip = 1.0 / jnp.maximum(cnt, onef)
            for j in range(CPG):
                acc = (_tree_sum([plsc.load_gather(big_v, [v + cvecs[j]])
                                  for v in iv0])
                       + _tree_sum([plsc.load_gather(big_v, [v + cvecs[j]])
                                    for v in iv1]))
                acc_v[p, j, pl.ds(rr, 16)] = (acc - n0f * t0s[j]) * recip

        odmas[p] = [pltpu.async_copy(
            acc_v.at[p, j],
            outT_hbm.at[EMB + cbase + j, pl.ds(r0, P2_CH)], osems[p])
            for j in range(CPG)]
        if ch + 2 < N2CH:
            idmas[p] = pltpu.async_copy(
                idsT_hbm.at[:, pl.ds(rbase + (ch + 2) * P2_CH, P2_CH)],
                ids_v.at[p], isems[p])
    for ds in odmas:
        for d in ds:
            d.wait()


def kernel(anime_ids, token_ids, name_table, text_table):
    aidx = anime_ids.astype(jnp.int32)
    idsT = token_ids.astype(jnp.int32).T
    nameF = name_table.T.reshape(NAMEF)
    textT = text_table.T

    mesh = plsc.VectorSubcoreMesh(core_axis_name="c", subcore_axis_name="s")
    run = functools.partial(
        pl.kernel, mesh=mesh,
        out_type=jax.ShapeDtypeStruct((2 * EMB, B), jnp.float32),
        compiler_params=pltpu.CompilerParams(
            needs_layout_passes=False, use_tc_tiling_on_sc=False),
        scratch_types=[
            pltpu.VMEM((VPAD,), jnp.float32),
            pltpu.VMEM((2, L, P2_CH), jnp.int32),
            pltpu.VMEM((2, P1_CH), jnp.int32),
            pltpu.VMEM((2, P1_CH), jnp.float32),
            pltpu.VMEM((2, CPG, P2_CH), jnp.float32),
            pltpu.SemaphoreType.DMA,
            pltpu.SemaphoreType.DMA,
            pltpu.SemaphoreType.DMA,
            pltpu.SemaphoreType.DMA,
            pltpu.SemaphoreType.DMA,
        ],
    )(_body)
    outT = run(aidx, idsT, nameF, textT)
    return outT.T
